# baseline (speedup 1.0000x reference)
import functools

import jax
import jax.numpy as jnp
from jax import lax
from jax.experimental import pallas as pl
from jax.experimental.pallas import tpu as pltpu
from jax.experimental.pallas import tpu_sc as plsc

BATCH = 16384
EMBED = 64
HIDDEN = 256
NUM_USERS = 1000000
NUM_MOVIES = 100000

NUM_CORES = 2
NUM_SUBCORES = 16
NUM_WORKERS = NUM_CORES * NUM_SUBCORES
B_PER_W = BATCH // NUM_WORKERS
CHUNK = 256

TC_RB = 64


def _pack_body(top_ref, bot_ref, o_ref):
    o_ref[...] = jnp.concatenate([top_ref[...], bot_ref[...]], axis=1)


def _pack_pairs(table, block_rows):
    half = table.shape[0] // 2
    grid = (half // block_rows,)
    return pl.pallas_call(
        _pack_body,
        grid=grid,
        in_specs=[
            pl.BlockSpec((block_rows, EMBED), lambda i: (i, 0)),
            pl.BlockSpec(
                (block_rows, EMBED), lambda i: (i + half // block_rows, 0)
            ),
        ],
        out_specs=pl.BlockSpec((block_rows, 2 * EMBED), lambda i: (i, 0)),
        out_shape=jax.ShapeDtypeStruct((half, 2 * EMBED), jnp.float32),
        compiler_params=pltpu.CompilerParams(
            dimension_semantics=("parallel",),
        ),
    )(table, table)


def _make_movie_gather_kernel():
    mesh = plsc.VectorSubcoreMesh(
        core_axis_name="c",
        subcore_axis_name="s",
        num_cores=NUM_CORES,
        num_subcores=NUM_SUBCORES,
    )

    @functools.partial(
        pl.kernel,
        mesh=mesh,
        out_type=jax.ShapeDtypeStruct((BATCH, 2 * EMBED), jnp.float32),
        scratch_types=[
            pltpu.VMEM((CHUNK,), jnp.int32),
            pltpu.VMEM((CHUNK, 2 * EMBED), jnp.float32),
            pltpu.SemaphoreType.DMA,
        ],
    )
    def gather_kernel(pairs_hbm, idx_hbm, out_hbm, idx_v, rows_v, sem):
        wid = lax.axis_index("s") * NUM_CORES + lax.axis_index("c")
        base = wid * B_PER_W
        for c in range(B_PER_W // CHUNK):
            base_c = base + c * CHUNK
            pltpu.sync_copy(idx_hbm.at[pl.ds(base_c, CHUNK)], idx_v)
            pltpu.async_copy(pairs_hbm.at[idx_v], rows_v, sem).wait()
            pltpu.sync_copy(rows_v, out_hbm.at[pl.ds(base_c, CHUNK)])

    return gather_kernel


@functools.lru_cache(maxsize=1)
def _get_movie_gather():
    return _make_movie_gather_kernel()


def _tc_gather_body(idx_smem, table_hbm, o_ref, sem):
    i = pl.program_id(0)
    cps = []
    for r in range(TC_RB):
        idx = idx_smem[i * TC_RB + r]
        cp = pltpu.make_async_copy(
            table_hbm.at[pl.ds(idx, 1)], o_ref.at[pl.ds(r, 1)], sem
        )
        cp.start()
        cps.append(cp)
    for cp in cps:
        cp.wait()


def _tc_gather(table, idx):
    grid_spec = pltpu.PrefetchScalarGridSpec(
        num_scalar_prefetch=1,
        grid=(BATCH // TC_RB,),
        in_specs=[pl.BlockSpec(memory_space=pltpu.MemorySpace.HBM)],
        out_specs=pl.BlockSpec((TC_RB, EMBED), lambda i, *_: (i, 0)),
        scratch_shapes=[pltpu.SemaphoreType.DMA],
    )
    return pl.pallas_call(
        _tc_gather_body,
        grid_spec=grid_spec,
        out_shape=jax.ShapeDtypeStruct((BATCH, EMBED), jnp.float32),
    )(idx, table)


def _mlp_body(
    u_ref, mp_ref, hm_ref, w1u_ref, w1m_ref, b1_ref, w2_ref, b2_ref, o_ref
):
    m = jnp.where(hm_ref[...] > 0, mp_ref[:, EMBED:], mp_ref[:, :EMBED])
    h = (
        jnp.dot(u_ref[...], w1u_ref[...], preferred_element_type=jnp.float32)
        + jnp.dot(m, w1m_ref[...], preferred_element_type=jnp.float32)
        + b1_ref[...]
    )
    h = jnp.maximum(h, 0.0)
    o_ref[...] = jnp.sum(h * w2_ref[...], axis=1, keepdims=True) + b2_ref[...]


def _mlp(u, mp, hm, w1u, w1m, b1_2d, w2, b2_2d, block_rows=2048):
    grid = (BATCH // block_rows,)
    return pl.pallas_call(
        _mlp_body,
        grid=grid,
        in_specs=[
            pl.BlockSpec((block_rows, EMBED), lambda i: (i, 0)),
            pl.BlockSpec((block_rows, 2 * EMBED), lambda i: (i, 0)),
            pl.BlockSpec((block_rows, 1), lambda i: (i, 0)),
            pl.BlockSpec((EMBED, HIDDEN), lambda i: (0, 0)),
            pl.BlockSpec((EMBED, HIDDEN), lambda i: (0, 0)),
            pl.BlockSpec((1, HIDDEN), lambda i: (0, 0)),
            pl.BlockSpec((1, HIDDEN), lambda i: (0, 0)),
            pl.BlockSpec((1, 1), lambda i: (0, 0)),
        ],
        out_specs=pl.BlockSpec((block_rows, 1), lambda i: (i, 0)),
        out_shape=jax.ShapeDtypeStruct((BATCH, 1), jnp.float32),
        compiler_params=pltpu.CompilerParams(
            dimension_semantics=("parallel",),
        ),
    )(u, mp, hm, w1u, w1m, b1_2d, w2, b2_2d)


@jax.jit
def kernel(users, movies, user_table, movie_table, W1, b1, W2, b2):
    users = users.astype(jnp.int32)
    movies = movies.astype(jnp.int32)
    mh = NUM_MOVIES // 2
    m_pair_idx = jnp.where(movies >= mh, movies - mh, movies)
    hm = (movies >= mh).astype(jnp.int32).reshape(-1, 1)
    movie_pairs = _pack_pairs(movie_table, 10000)
    mp_rows = _get_movie_gather()(movie_pairs, m_pair_idx)
    u_rows = _tc_gather(user_table, users)
    w1t = W1.T
    w1u = w1t[:EMBED]
    w1m = w1t[EMBED:]
    b1_2d = b1.reshape(1, HIDDEN)
    b2_2d = b2.reshape(1, 1)
    return _mlp(u_rows, mp_rows, hm, w1u, w1m, b1_2d, W2, b2_2d)


# speedup vs baseline: 1.2365x; 1.2365x over previous
import functools

import jax
import jax.numpy as jnp
from jax import lax
from jax.experimental import pallas as pl
from jax.experimental.pallas import tpu as pltpu
from jax.experimental.pallas import tpu_sc as plsc

BATCH = 16384
EMBED = 64
HIDDEN = 256
NUM_USERS = 1000000
NUM_MOVIES = 100000

NUM_CORES = 2
NUM_SUBCORES = 16
NUM_WORKERS = NUM_CORES * NUM_SUBCORES
B_PER_W = BATCH // NUM_WORKERS
CHUNK = 256

TC_RB = 256
U_SPLIT = BATCH // 2
U_PER_W = U_SPLIT // NUM_WORKERS


def _pack_body(top_ref, bot_ref, o_ref):
    o_ref[...] = jnp.concatenate([top_ref[...], bot_ref[...]], axis=1)


def _pack_pairs(table, block_rows):
    half = table.shape[0] // 2
    grid = (half // block_rows,)
    return pl.pallas_call(
        _pack_body,
        grid=grid,
        in_specs=[
            pl.BlockSpec((block_rows, EMBED), lambda i: (i, 0)),
            pl.BlockSpec(
                (block_rows, EMBED), lambda i: (i + half // block_rows, 0)
            ),
        ],
        out_specs=pl.BlockSpec((block_rows, 2 * EMBED), lambda i: (i, 0)),
        out_shape=jax.ShapeDtypeStruct((half, 2 * EMBED), jnp.float32),
        compiler_params=pltpu.CompilerParams(
            dimension_semantics=("parallel",),
        ),
    )(table, table)


def _make_movie_gather_kernel():
    mesh = plsc.VectorSubcoreMesh(
        core_axis_name="c",
        subcore_axis_name="s",
        num_cores=NUM_CORES,
        num_subcores=NUM_SUBCORES,
    )

    @functools.partial(
        pl.kernel,
        mesh=mesh,
        out_type=(
            jax.ShapeDtypeStruct((BATCH, 2 * EMBED), jnp.float32),
            jax.ShapeDtypeStruct((U_SPLIT, EMBED), jnp.float32),
        ),
        scratch_types=[
            pltpu.VMEM((CHUNK,), jnp.int32),
            pltpu.VMEM((U_PER_W,), jnp.int32),
            pltpu.VMEM((CHUNK, 2 * EMBED), jnp.float32),
            pltpu.SemaphoreType.DMA,
            pltpu.SemaphoreType.DMA,
        ],
    )
    def gather_kernel(
        pairs_hbm,
        idx_hbm,
        user_table_hbm,
        users_hbm,
        out_hbm,
        out_u_hbm,
        idx_v,
        idx_u_v,
        rows_v,
        sem,
        sem_u,
    ):
        wid = lax.axis_index("s") * NUM_CORES + lax.axis_index("c")
        ubase = wid * U_PER_W
        pltpu.sync_copy(users_hbm.at[pl.ds(U_SPLIT + ubase, U_PER_W)], idx_u_v)

        @pl.loop(0, U_PER_W)
        def _(i):
            iu = idx_u_v[pl.ds(i, 1)][0]
            pltpu.async_copy(
                user_table_hbm.at[pl.ds(iu, 1)],
                out_u_hbm.at[pl.ds(ubase + i, 1)],
                sem_u,
            )

        base = wid * B_PER_W
        for c in range(B_PER_W // CHUNK):
            base_c = base + c * CHUNK
            pltpu.sync_copy(idx_hbm.at[pl.ds(base_c, CHUNK)], idx_v)
            pltpu.async_copy(pairs_hbm.at[idx_v], rows_v, sem).wait()
            pltpu.sync_copy(rows_v, out_hbm.at[pl.ds(base_c, CHUNK)])

        pltpu.make_async_copy(
            user_table_hbm.at[pl.ds(0, U_PER_W)],
            out_u_hbm.at[pl.ds(ubase, U_PER_W)],
            sem_u,
        ).wait()

    return gather_kernel


@functools.lru_cache(maxsize=1)
def _get_movie_gather():
    return _make_movie_gather_kernel()


def _tc_gather_body(idx_smem, table_hbm, o_ref, sem):
    i = pl.program_id(0)
    cps = []
    for r in range(TC_RB):
        idx = idx_smem[i * TC_RB + r]
        cp = pltpu.make_async_copy(
            table_hbm.at[pl.ds(idx, 1)], o_ref.at[pl.ds(r, 1)], sem
        )
        cp.start()
        cps.append(cp)
    for cp in cps:
        cp.wait()


def _tc_gather(table, idx):
    grid_spec = pltpu.PrefetchScalarGridSpec(
        num_scalar_prefetch=1,
        grid=(U_SPLIT // TC_RB,),
        in_specs=[pl.BlockSpec(memory_space=pltpu.MemorySpace.HBM)],
        out_specs=pl.BlockSpec((TC_RB, EMBED), lambda i, *_: (i, 0)),
        scratch_shapes=[pltpu.SemaphoreType.DMA],
    )
    return pl.pallas_call(
        _tc_gather_body,
        grid_spec=grid_spec,
        out_shape=jax.ShapeDtypeStruct((U_SPLIT, EMBED), jnp.float32),
    )(idx, table)


def _mlp_body(
    u_ref, mp_ref, hm_ref, w1u_ref, w1m_ref, b1_ref, w2_ref, b2_ref, o_ref
):
    m = jnp.where(hm_ref[...] > 0, mp_ref[:, EMBED:], mp_ref[:, :EMBED])
    h = (
        jnp.dot(u_ref[...], w1u_ref[...], preferred_element_type=jnp.float32)
        + jnp.dot(m, w1m_ref[...], preferred_element_type=jnp.float32)
        + b1_ref[...]
    )
    h = jnp.maximum(h, 0.0)
    o_ref[...] = jnp.sum(h * w2_ref[...], axis=1, keepdims=True) + b2_ref[...]


def _mlp(u, mp, hm, w1u, w1m, b1_2d, w2, b2_2d, block_rows=2048):
    grid = (BATCH // block_rows,)
    return pl.pallas_call(
        _mlp_body,
        grid=grid,
        in_specs=[
            pl.BlockSpec((block_rows, EMBED), lambda i: (i, 0)),
            pl.BlockSpec((block_rows, 2 * EMBED), lambda i: (i, 0)),
            pl.BlockSpec((block_rows, 1), lambda i: (i, 0)),
            pl.BlockSpec((EMBED, HIDDEN), lambda i: (0, 0)),
            pl.BlockSpec((EMBED, HIDDEN), lambda i: (0, 0)),
            pl.BlockSpec((1, HIDDEN), lambda i: (0, 0)),
            pl.BlockSpec((1, HIDDEN), lambda i: (0, 0)),
            pl.BlockSpec((1, 1), lambda i: (0, 0)),
        ],
        out_specs=pl.BlockSpec((block_rows, 1), lambda i: (i, 0)),
        out_shape=jax.ShapeDtypeStruct((BATCH, 1), jnp.float32),
        compiler_params=pltpu.CompilerParams(
            dimension_semantics=("parallel",),
        ),
    )(u, mp, hm, w1u, w1m, b1_2d, w2, b2_2d)


@jax.jit
def kernel(users, movies, user_table, movie_table, W1, b1, W2, b2):
    users = users.astype(jnp.int32)
    movies = movies.astype(jnp.int32)
    mh = NUM_MOVIES // 2
    m_pair_idx = jnp.where(movies >= mh, movies - mh, movies)
    hm = (movies >= mh).astype(jnp.int32).reshape(-1, 1)
    movie_pairs = _pack_pairs(movie_table, 10000)
    mp_rows, u_sc = _get_movie_gather()(movie_pairs, m_pair_idx, user_table, users)
    u_tc = _tc_gather(user_table, users[:U_SPLIT])
    u_rows = jnp.concatenate([u_tc, u_sc], axis=0)
    w1t = W1.T
    w1u = w1t[:EMBED]
    w1m = w1t[EMBED:]
    b1_2d = b1.reshape(1, HIDDEN)
    b2_2d = b2.reshape(1, 1)
    return _mlp(u_rows, mp_rows, hm, w1u, w1m, b1_2d, W2, b2_2d)
